# baseline (speedup 1.0000x reference)
import functools

import jax
import jax.numpy as jnp
from jax.experimental import pallas as pl


def _mm_kernel(x_ref, w_ref, o_ref):
    o_ref[...] = jnp.dot(x_ref[...], w_ref[...], preferred_element_type=jnp.float32)


def _matmul(x, w, bn=1000):
    M, K = x.shape
    _, N = w.shape
    return pl.pallas_call(
        _mm_kernel,
        grid=(M // bn,),
        in_specs=[
            pl.BlockSpec((bn, K), lambda i: (i, 0)),
            pl.BlockSpec((K, N), lambda i: (0, 0)),
        ],
        out_specs=pl.BlockSpec((bn, N), lambda i: (i, 0)),
        out_shape=jax.ShapeDtypeStruct((M, N), jnp.float32),
    )(x, w)


def _gat_layer(x, src, dst, W, a_src, a_dst, b, n_nodes):
    h = _matmul(x, W)
    s_src = h @ a_src
    s_dst = h @ a_dst
    e = jax.nn.leaky_relu(s_src[src] + s_dst[dst], negative_slope=0.2)
    m = jax.ops.segment_max(e, dst, num_segments=n_nodes)
    ex = jnp.exp(e - m[dst])
    denom = jax.ops.segment_sum(ex, dst, num_segments=n_nodes)
    alpha = ex / (denom[dst] + 1e-16)
    out = jax.ops.segment_sum(alpha[:, None] * h[src], dst, num_segments=n_nodes)
    return jax.nn.relu(out + b)


def kernel(x, edge_index, W0, a0_src, a0_dst, b0, W1, a1_src, a1_dst, b1,
           W2, a2_src, a2_dst, b2, Wm1, bm1, Wm2, bm2):
    n_nodes = x.shape[0]
    src = edge_index[0]
    dst = edge_index[1]
    h = _gat_layer(x, src, dst, W0, a0_src, a0_dst, b0, n_nodes)
    h = _gat_layer(h, src, dst, W1, a1_src, a1_dst, b1, n_nodes)
    h = _gat_layer(h, src, dst, W2, a2_src, a2_dst, b2, n_nodes)
    h = jax.nn.relu(_matmul(h, Wm1) + bm1)
    return _matmul(h, Wm2) + bm2


# speedup vs baseline: 10.5739x; 10.5739x over previous
import functools

import jax
import jax.numpy as jnp
from jax import lax
from jax.experimental import pallas as pl
from jax.experimental.pallas import tpu as pltpu
from jax.experimental.pallas import tpu_sc as plsc

NC = 2
NS = 16
NW = NC * NS
L = 16

NEG_SLOPE = 0.2


def _leaky(z):
    return jnp.where(z >= 0, z, z * NEG_SLOPE)


def _tcm0_body(x_ref, w_ref, a8_ref, h_ref, s8_ref, m_ref):
    i = pl.program_id(0)
    h = jnp.dot(x_ref[...], w_ref[...], preferred_element_type=jnp.float32)
    h_ref[...] = h
    s8 = jnp.dot(h, a8_ref[...], preferred_element_type=jnp.float32)
    s8_ref[...] = s8
    bm = jnp.full((1, 8), jnp.max(s8[:, 0]), jnp.float32)

    @pl.when(i == 0)
    def _():
        m_ref[...] = bm

    @pl.when(i > 0)
    def _():
        m_ref[...] = jnp.maximum(m_ref[...], bm)


def _tcmA_body(p0_ref, p1_ref, b_ref, w_ref, a8_ref, h_ref, s8_ref, m_ref):
    i = pl.program_id(0)
    x = jax.nn.relu(p0_ref[...] + p1_ref[...] + b_ref[...])
    h = jnp.dot(x, w_ref[...], preferred_element_type=jnp.float32)
    h_ref[...] = h
    s8 = jnp.dot(h, a8_ref[...], preferred_element_type=jnp.float32)
    s8_ref[...] = s8
    bm = jnp.full((1, 8), jnp.max(s8[:, 0]), jnp.float32)

    @pl.when(i == 0)
    def _():
        m_ref[...] = bm

    @pl.when(i > 0)
    def _():
        m_ref[...] = jnp.maximum(m_ref[...], bm)


def _mlp_body(p0_ref, p1_ref, b_ref, wm1_ref, bm1_ref, wm2_ref, bm2_ref, o_ref):
    x = jax.nn.relu(p0_ref[...] + p1_ref[...] + b_ref[...])
    t = jax.nn.relu(jnp.dot(x, wm1_ref[...], preferred_element_type=jnp.float32)
                    + bm1_ref[...])
    o_ref[...] = jnp.dot(t, wm2_ref[...], preferred_element_type=jnp.float32) \
        + bm2_ref[...]


def _tc_layer0(x, W, a8, bn=1000):
    n, d_in = x.shape
    d_h = W.shape[1]
    return pl.pallas_call(
        _tcm0_body,
        grid=(n // bn,),
        in_specs=[
            pl.BlockSpec((bn, d_in), lambda i: (i, 0)),
            pl.BlockSpec((d_in, d_h), lambda i: (0, 0)),
            pl.BlockSpec((d_h, 8), lambda i: (0, 0)),
        ],
        out_specs=[
            pl.BlockSpec((bn, d_h), lambda i: (i, 0)),
            pl.BlockSpec((bn, 8), lambda i: (i, 0)),
            pl.BlockSpec((1, 8), lambda i: (0, 0)),
        ],
        out_shape=[
            jax.ShapeDtypeStruct((n, d_h), jnp.float32),
            jax.ShapeDtypeStruct((n, 8), jnp.float32),
            jax.ShapeDtypeStruct((1, 8), jnp.float32),
        ],
    )(x, W, a8)


def _tc_layerA(p0, p1, b, W, a8, bn=1000):
    n, d_h = p0.shape
    return pl.pallas_call(
        _tcmA_body,
        grid=(n // bn,),
        in_specs=[
            pl.BlockSpec((bn, d_h), lambda i: (i, 0)),
            pl.BlockSpec((bn, d_h), lambda i: (i, 0)),
            pl.BlockSpec((1, d_h), lambda i: (0, 0)),
            pl.BlockSpec((d_h, d_h), lambda i: (0, 0)),
            pl.BlockSpec((d_h, 8), lambda i: (0, 0)),
        ],
        out_specs=[
            pl.BlockSpec((bn, d_h), lambda i: (i, 0)),
            pl.BlockSpec((bn, 8), lambda i: (i, 0)),
            pl.BlockSpec((1, 8), lambda i: (0, 0)),
        ],
        out_shape=[
            jax.ShapeDtypeStruct((n, d_h), jnp.float32),
            jax.ShapeDtypeStruct((n, 8), jnp.float32),
            jax.ShapeDtypeStruct((1, 8), jnp.float32),
        ],
    )(p0, p1, b.reshape(1, -1), W, a8)


def _tc_mlp(p0, p1, b, Wm1, bm1, Wm2, bm2, bn=1000):
    n, d_h = p0.shape
    d_mlp = Wm1.shape[1]
    n_lab = Wm2.shape[1]
    return pl.pallas_call(
        _mlp_body,
        grid=(n // bn,),
        in_specs=[
            pl.BlockSpec((bn, d_h), lambda i: (i, 0)),
            pl.BlockSpec((bn, d_h), lambda i: (i, 0)),
            pl.BlockSpec((1, d_h), lambda i: (0, 0)),
            pl.BlockSpec((d_h, d_mlp), lambda i: (0, 0)),
            pl.BlockSpec((1, d_mlp), lambda i: (0, 0)),
            pl.BlockSpec((d_mlp, n_lab), lambda i: (0, 0)),
            pl.BlockSpec((1, n_lab), lambda i: (0, 0)),
        ],
        out_specs=pl.BlockSpec((bn, n_lab), lambda i: (i, 0)),
        out_shape=jax.ShapeDtypeStruct((n, n_lab), jnp.float32),
    )(p0, p1, b.reshape(1, -1), Wm1, bm1.reshape(1, -1), Wm2, bm2.reshape(1, -1))


def _sc_mesh():
    return plsc.VectorSubcoreMesh(core_axis_name="c", subcore_axis_name="s",
                                  num_cores=NC, num_subcores=NS)


def _make_a1(n_nodes, n_edges, kchunks, n_pad):
    ept = kchunks * 128

    @functools.partial(
        pl.kernel,
        out_type=[
            jax.ShapeDtypeStruct((NW, kchunks, 128), jnp.float32),
            jax.ShapeDtypeStruct((NC, n_pad), jnp.float32),
        ],
        mesh=_sc_mesh(),
        compiler_params=pltpu.CompilerParams(needs_layout_passes=False),
        scratch_types=[
            pltpu.VMEM((kchunks, 128), jnp.int32),
            pltpu.VMEM((kchunks, 128), jnp.int32),
            pltpu.VMEM((kchunks, 128), jnp.float32),
            pltpu.VMEM((n_nodes,), jnp.float32),
            pltpu.VMEM((n_nodes,), jnp.float32),
            pltpu.VMEM((L,), jnp.float32),
            pltpu.VMEM((n_pad,), jnp.float32),
            pltpu.VMEM_SHARED((n_pad,), jnp.float32),
        ],
    )
    def a1(src_hbm, dst_hbm, ssrc_hbm, sdst_hbm, mvec_hbm,
           ex_hbm, dpart_hbm,
           idx_s, idx_d, exbuf, ssrc_t, sdst_t, mvec_t, zbuf, dshared):
        cid = lax.axis_index("c")
        sid = lax.axis_index("s")
        wid = sid * NC + cid

        pltpu.sync_copy(ssrc_hbm, ssrc_t)
        pltpu.sync_copy(sdst_hbm, sdst_t)
        pltpu.sync_copy(mvec_hbm, mvec_t)
        pltpu.sync_copy(src_hbm.at[wid], idx_s)
        pltpu.sync_copy(dst_hbm.at[wid], idx_d)

        @pl.when(sid == 0)
        def _():
            @pl.loop(0, n_pad // L)
            def _(i):
                zbuf[pl.ds(i * L, L)] = jnp.zeros((L,), jnp.float32)
            pltpu.sync_copy(zbuf, dshared)

        plsc.subcore_barrier()

        mv = mvec_t[...]
        ebase = wid * ept

        @pl.loop(0, kchunks)
        def _(j):
            for g in range(128 // L):
                si = idx_s[j, pl.ds(g * L, L)]
                di = idx_d[j, pl.ds(g * L, L)]
                vs = plsc.load_gather(ssrc_t, [si])
                vd = plsc.load_gather(sdst_t, [di])
                e = _leaky(vs + vd)
                mb = _leaky(vd + mv)
                ex = jnp.exp(e - mb)
                eid = ebase + j * 128 + g * L + lax.iota(jnp.int32, L)
                ex = jnp.where(eid < n_edges, ex, 0.0)
                exbuf[j, pl.ds(g * L, L)] = ex
            pltpu.sync_copy(exbuf.at[j], dshared.at[idx_d.at[j]], add=True)

        pltpu.sync_copy(exbuf, ex_hbm.at[wid])
        plsc.subcore_barrier()

        @pl.when(sid == 0)
        def _():
            pltpu.sync_copy(dshared, dpart_hbm.at[cid])

    return a1


def _make_a2(n_nodes, kchunks, d_h, n_pad):
    rows_per_tile = n_pad // NS
    full128 = rows_per_tile // 128
    tail = rows_per_tile - full128 * 128

    @functools.partial(
        pl.kernel,
        out_type=jax.ShapeDtypeStruct((NC * n_pad, d_h), jnp.float32),
        mesh=_sc_mesh(),
        compiler_params=pltpu.CompilerParams(needs_layout_passes=False),
        scratch_types=[
            pltpu.VMEM((kchunks, 128), jnp.int32),
            pltpu.VMEM((kchunks, 128), jnp.int32),
            pltpu.VMEM((kchunks, 128), jnp.float32),
            pltpu.VMEM((n_pad,), jnp.float32),
            pltpu.VMEM((2048,), jnp.float32),
            pltpu.VMEM((128,), jnp.float32),
            pltpu.VMEM((128, d_h), jnp.float32),
            pltpu.VMEM_SHARED((n_pad, d_h), jnp.float32),
        ],
    )
    def a2(src_hbm, dst_hbm, ex_hbm, dpart_hbm, h_hbm,
           outp_hbm,
           idx_s, idx_d, exbuf, dtab, dtab2, alpha_c, rowbuf, oshared):
        cid = lax.axis_index("c")
        sid = lax.axis_index("s")
        wid = sid * NC + cid

        pltpu.sync_copy(src_hbm.at[wid], idx_s)
        pltpu.sync_copy(dst_hbm.at[wid], idx_d)
        pltpu.sync_copy(ex_hbm.at[wid], exbuf)
        pltpu.sync_copy(dpart_hbm.at[0], dtab)
        n_chunks2 = (n_pad + 2047) // 2048
        for c in range(n_chunks2):
            lo = c * 2048
            sz = min(2048, n_pad - lo)
            pltpu.sync_copy(dpart_hbm.at[1, pl.ds(lo, sz)], dtab2.at[pl.ds(0, sz)])

            @pl.loop(0, sz // L)
            def _(i):
                dtab[pl.ds(lo + i * L, L)] = (dtab[pl.ds(lo + i * L, L)]
                                              + dtab2[pl.ds(i * L, L)])

        @pl.loop(0, 128)
        def _(i):
            for q in range(d_h // L):
                rowbuf[i, pl.ds(q * L, L)] = jnp.zeros((L,), jnp.float32)
        rbase = sid * rows_per_tile
        for k in range(full128):
            pltpu.sync_copy(rowbuf, oshared.at[pl.ds(rbase + k * 128, 128)])
        if tail:
            pltpu.sync_copy(rowbuf.at[pl.ds(0, tail)],
                            oshared.at[pl.ds(rbase + full128 * 128, tail)])

        plsc.subcore_barrier()

        @pl.loop(0, kchunks)
        def _(j):
            for g in range(128 // L):
                di = idx_d[j, pl.ds(g * L, L)]
                dv = plsc.load_gather(dtab, [di])
                exv = exbuf[j, pl.ds(g * L, L)]
                alpha_c[pl.ds(g * L, L)] = exv / (dv + 1e-16)
            pltpu.sync_copy(h_hbm.at[idx_s.at[j]], rowbuf)

            @pl.loop(0, 128)
            def _(e):
                av = plsc.load_gather(alpha_c, [jnp.full((L,), e, jnp.int32)])
                for q in range(d_h // L):
                    rowbuf[e, pl.ds(q * L, L)] = rowbuf[e, pl.ds(q * L, L)] * av

            pltpu.sync_copy(rowbuf, oshared.at[idx_d.at[j]], add=True)

        plsc.subcore_barrier()
        pltpu.sync_copy(oshared.at[pl.ds(rbase, rows_per_tile)],
                        outp_hbm.at[pl.ds(cid * n_pad + rbase, rows_per_tile)])

    return a2


def _a8(a_src, a_dst):
    d = a_src.shape[0]
    a8 = jnp.zeros((d, 8), jnp.float32)
    return a8.at[:, 0].set(a_src).at[:, 1].set(a_dst)


def kernel(x, edge_index, W0, a0_src, a0_dst, b0, W1, a1_src, a1_dst, b1,
           W2, a2_src, a2_dst, b2, Wm1, bm1, Wm2, bm2):
    n_nodes = x.shape[0]
    n_edges = edge_index.shape[1]
    d_h = W0.shape[1]

    e_pad = ((n_edges + NW * 128 - 1) // (NW * 128)) * (NW * 128)
    kchunks = e_pad // (NW * 128)
    pad = e_pad - n_edges
    src3 = jnp.concatenate(
        [edge_index[0], jnp.zeros((pad,), jnp.int32)]).reshape(NW, kchunks, 128)
    dst3 = jnp.concatenate(
        [edge_index[1], jnp.zeros((pad,), jnp.int32)]).reshape(NW, kchunks, 128)

    n_pad = NS * 8 * ((n_nodes + NS * 8 - 1) // (NS * 8))
    a1_fn = _make_a1(n_nodes, n_edges, kchunks, n_pad)
    a2_fn = _make_a2(n_nodes, kchunks, d_h, n_pad)

    def edge_phase(h, s8, mstat):
        s_src = s8[:, 0]
        s_dst = s8[:, 1]
        mvec = jnp.full((L,), mstat[0, 0], jnp.float32)
        ex, dparts = a1_fn(src3, dst3, s_src, s_dst, mvec)
        outp = a2_fn(src3, dst3, ex, dparts, h)
        return outp[:n_nodes], outp[n_pad:n_pad + n_nodes]

    h, s8, mstat = _tc_layer0(x, W0, _a8(a0_src, a0_dst))
    p0, p1 = edge_phase(h, s8, mstat)
    h, s8, mstat = _tc_layerA(p0, p1, b0, W1, _a8(a1_src, a1_dst))
    p0, p1 = edge_phase(h, s8, mstat)
    h, s8, mstat = _tc_layerA(p0, p1, b1, W2, _a8(a2_src, a2_dst))
    p0, p1 = edge_phase(h, s8, mstat)
    return _tc_mlp(p0, p1, b2, Wm1, bm1, Wm2, bm2)
